# baseline (speedup 1.0000x reference)
import functools

import jax
import jax.numpy as jnp
from jax import lax
from jax.experimental import pallas as pl
from jax.experimental.pallas import tpu as pltpu
from jax.experimental.pallas import tpu_sc as plsc

_RADIUS = 0.2
_NSAMPLE = 32
_NC = 2
_NS = 16
_NW = _NC * _NS
_L = 16


def _bf(x):
    u = plsc.bitcast(x, jnp.uint32)
    lsb = (u >> jnp.full(x.shape, 16, jnp.uint32)) & jnp.full(
        x.shape, 1, jnp.uint32)
    r = (u + jnp.full(x.shape, 0x7FFF, jnp.uint32) + lsb) & jnp.full(
        x.shape, 0xFFFF0000, jnp.uint32)
    return plsc.bitcast(r, jnp.float32)


def _sc_body(coord_hbm, samples_hbm, out_hbm, cvm, svm, ovm,
             *, n, s, rows_per_w):
    wid = lax.axis_index("s") * _NC + lax.axis_index("c")
    row0 = wid * rows_per_w
    batch = row0 // s
    srow0 = row0 % s
    r2 = _RADIUS * _RADIUS

    pltpu.sync_copy(coord_hbm.at[batch], cvm)
    pltpu.sync_copy(samples_hbm.at[batch], svm)

    lanes = lax.iota(jnp.int32, 16)
    nv = jnp.full((_L,), n, jnp.int32)
    onesv = jnp.full((_L,), 1, jnp.int32)

    def row_body(i, carry):
        sidx = jnp.full((_L,), srow0 + i, jnp.int32)
        sx = plsc.load_gather(svm, [sidx])
        sy = plsc.load_gather(svm, [sidx + jnp.full((_L,), s, jnp.int32)])
        sz = plsc.load_gather(svm, [sidx + jnp.full((_L,), 2 * s, jnp.int32)])
        s2 = sx * sx + sy * sy + sz * sz
        sxb, syb, szb = _bf(sx), _bf(sy), _bf(sz)
        base = i * _NSAMPLE
        ovm[pl.ds(base, _L)] = nv
        ovm[pl.ds(base + _L, _L)] = nv

        def step(jb, have_v):
            cx = cvm[0, pl.ds(jb, _L)]
            cy = cvm[1, pl.ds(jb, _L)]
            cz = cvm[2, pl.ds(jb, _L)]
            mm = sxb * _bf(cx)
            mm = mm + syb * _bf(cy)
            mm = mm + szb * _bf(cz)
            c2 = cx * cx + cy * cy + cz * cz
            d = (-2.0 * mm + s2) + c2
            msk = d <= r2
            mi = jnp.where(msk, 1, 0).astype(jnp.int32)
            ranks = plsc.cumsum(mi)
            pos = have_v + ranks
            wmask = msk & (pos <= _NSAMPLE)
            vals = jnp.full((_L,), jb, jnp.int32) + lanes
            idx = pos - 1 + jnp.full((_L,), base, jnp.int32)
            plsc.store_scatter(ovm, [idx], vals, mask=wmask)
            cnt = plsc.all_reduce_population_count(msk)
            return have_v + cnt

        def cond(st):
            jb, _, done = st
            return jnp.logical_and(jb < n, jnp.logical_not(done))

        def body(st):
            jb, have_v, _ = st
            have_v = step(jb, have_v)
            have_v = step(jb + _L, have_v)
            done = jnp.any(have_v >= _NSAMPLE)
            return (jb + 2 * _L, have_v, done)

        lax.while_loop(cond, body,
                       (jnp.int32(0), jnp.zeros((_L,), jnp.int32), False))

        first = plsc.load_gather(ovm, [jnp.full((_L,), base, jnp.int32)])
        o1 = ovm[pl.ds(base, _L)]
        o2 = ovm[pl.ds(base + _L, _L)]
        ovm[pl.ds(base, _L)] = jnp.where(o1 == nv, first, o1)
        ovm[pl.ds(base + _L, _L)] = jnp.where(o2 == nv, first, o2)
        return carry

    lax.fori_loop(0, rows_per_w, row_body, 0)
    pltpu.sync_copy(ovm, out_hbm.at[pl.ds(row0 * _NSAMPLE,
                                          rows_per_w * _NSAMPLE)])


def kernel(coord, samples):
    b, n, _ = coord.shape
    s = samples.shape[1]
    rows_per_w = (b * s) // _NW

    coord_t = jnp.transpose(coord, (0, 2, 1))
    samples_t = jnp.transpose(samples, (0, 2, 1)).reshape(b, 3 * s)

    mesh = plsc.VectorSubcoreMesh(core_axis_name="c", subcore_axis_name="s",
                                  num_cores=_NC, num_subcores=_NS)
    out = pl.kernel(
        functools.partial(_sc_body, n=n, s=s, rows_per_w=rows_per_w),
        out_type=jax.ShapeDtypeStruct((b * s * _NSAMPLE,), jnp.int32),
        mesh=mesh,
        compiler_params=pltpu.CompilerParams(needs_layout_passes=False),
        scratch_types=[
            pltpu.VMEM((3, n), jnp.float32),
            pltpu.VMEM((3 * s,), jnp.float32),
            pltpu.VMEM((rows_per_w * _NSAMPLE,), jnp.int32),
        ],
    )(coord_t, samples_t)
    return out.reshape(b, s, _NSAMPLE)


# speedup vs baseline: 1.2593x; 1.2593x over previous
import functools

import jax
import jax.numpy as jnp
from jax import lax
from jax.experimental import pallas as pl
from jax.experimental.pallas import tpu as pltpu
from jax.experimental.pallas import tpu_sc as plsc

_RADIUS = 0.2
_NSAMPLE = 32
_NC = 2
_NS = 16
_NW = _NC * _NS
_L = 16


def _bf(x):
    u = plsc.bitcast(x, jnp.uint32)
    lsb = (u >> jnp.full(x.shape, 16, jnp.uint32)) & jnp.full(
        x.shape, 1, jnp.uint32)
    r = (u + jnp.full(x.shape, 0x7FFF, jnp.uint32) + lsb) & jnp.full(
        x.shape, 0xFFFF0000, jnp.uint32)
    return plsc.bitcast(r, jnp.float32)


def _sc_body(coord_hbm, samples_hbm, out_hbm, cvm, svm, ovm,
             cxb, cyb, czb, c2v, *, n, s, rows_per_w):
    wid = lax.axis_index("s") * _NC + lax.axis_index("c")
    row0 = wid * rows_per_w
    batch = row0 // s
    srow0 = row0 % s
    r2 = _RADIUS * _RADIUS

    pltpu.sync_copy(coord_hbm.at[batch], cvm)
    pltpu.sync_copy(samples_hbm.at[batch], svm)

    lanes = lax.iota(jnp.int32, 16)
    nv = jnp.full((_L,), n, jnp.int32)
    onesv = jnp.full((_L,), 1, jnp.int32)

    def pre_body(t, carry):
        jb = t * _L
        cx = cvm[0, pl.ds(jb, _L)]
        cy = cvm[1, pl.ds(jb, _L)]
        cz = cvm[2, pl.ds(jb, _L)]
        cxb[pl.ds(jb, _L)] = _bf(cx)
        cyb[pl.ds(jb, _L)] = _bf(cy)
        czb[pl.ds(jb, _L)] = _bf(cz)
        c2v[pl.ds(jb, _L)] = cx * cx + cy * cy + cz * cz
        return carry

    lax.fori_loop(0, n // _L, pre_body, 0)

    def row_body(i, carry):
        sidx = jnp.full((_L,), srow0 + i, jnp.int32)
        sx = plsc.load_gather(svm, [sidx])
        sy = plsc.load_gather(svm, [sidx + jnp.full((_L,), s, jnp.int32)])
        sz = plsc.load_gather(svm, [sidx + jnp.full((_L,), 2 * s, jnp.int32)])
        s2 = sx * sx + sy * sy + sz * sz
        sxb, syb, szb = _bf(sx), _bf(sy), _bf(sz)
        base = i * _NSAMPLE
        ovm[pl.ds(base, _L)] = nv
        ovm[pl.ds(base + _L, _L)] = nv

        def step(jb, have_v):
            mm = sxb * cxb[pl.ds(jb, _L)]
            mm = mm + syb * cyb[pl.ds(jb, _L)]
            mm = mm + szb * czb[pl.ds(jb, _L)]
            d = (-2.0 * mm + s2) + c2v[pl.ds(jb, _L)]
            msk = d <= r2
            mi = jnp.where(msk, 1, 0).astype(jnp.int32)
            ranks = plsc.cumsum(mi)
            pos = have_v + ranks
            wmask = msk & (pos <= _NSAMPLE)
            vals = jnp.full((_L,), jb, jnp.int32) + lanes
            idx = pos - 1 + jnp.full((_L,), base, jnp.int32)
            plsc.store_scatter(ovm, [idx], vals, mask=wmask)
            cnt = plsc.all_reduce_population_count(msk)
            return have_v + cnt

        def cond(st):
            jb, _, done = st
            return jnp.logical_and(jb < n, jnp.logical_not(done))

        def body(st):
            jb, have_v, _ = st
            for u in range(4):
                have_v = step(jb + u * _L, have_v)
            done = jnp.any(have_v >= _NSAMPLE)
            return (jb + 4 * _L, have_v, done)

        lax.while_loop(cond, body,
                       (jnp.int32(0), jnp.zeros((_L,), jnp.int32), False))

        first = plsc.load_gather(ovm, [jnp.full((_L,), base, jnp.int32)])
        o1 = ovm[pl.ds(base, _L)]
        o2 = ovm[pl.ds(base + _L, _L)]
        ovm[pl.ds(base, _L)] = jnp.where(o1 == nv, first, o1)
        ovm[pl.ds(base + _L, _L)] = jnp.where(o2 == nv, first, o2)
        return carry

    lax.fori_loop(0, rows_per_w, row_body, 0)
    pltpu.sync_copy(ovm, out_hbm.at[pl.ds(row0 * _NSAMPLE,
                                          rows_per_w * _NSAMPLE)])


def kernel(coord, samples):
    b, n, _ = coord.shape
    s = samples.shape[1]
    rows_per_w = (b * s) // _NW

    coord_t = jnp.transpose(coord, (0, 2, 1))
    samples_t = jnp.transpose(samples, (0, 2, 1)).reshape(b, 3 * s)

    mesh = plsc.VectorSubcoreMesh(core_axis_name="c", subcore_axis_name="s",
                                  num_cores=_NC, num_subcores=_NS)
    out = pl.kernel(
        functools.partial(_sc_body, n=n, s=s, rows_per_w=rows_per_w),
        out_type=jax.ShapeDtypeStruct((b * s * _NSAMPLE,), jnp.int32),
        mesh=mesh,
        compiler_params=pltpu.CompilerParams(needs_layout_passes=False),
        scratch_types=[
            pltpu.VMEM((3, n), jnp.float32),
            pltpu.VMEM((3 * s,), jnp.float32),
            pltpu.VMEM((rows_per_w * _NSAMPLE,), jnp.int32),
            pltpu.VMEM((n,), jnp.float32),
            pltpu.VMEM((n,), jnp.float32),
            pltpu.VMEM((n,), jnp.float32),
            pltpu.VMEM((n,), jnp.float32),
        ],
    )(coord_t, samples_t)
    return out.reshape(b, s, _NSAMPLE)


# speedup vs baseline: 1.7803x; 1.4137x over previous
import functools

import jax
import jax.numpy as jnp
from jax import lax
from jax.experimental import pallas as pl
from jax.experimental.pallas import tpu as pltpu
from jax.experimental.pallas import tpu_sc as plsc

_RADIUS = 0.2
_NSAMPLE = 32
_NC = 2
_NS = 16
_NW = _NC * _NS
_L = 16


def _bf(x):
    u = plsc.bitcast(x, jnp.uint32)
    lsb = (u >> jnp.full(x.shape, 16, jnp.uint32)) & jnp.full(
        x.shape, 1, jnp.uint32)
    r = (u + jnp.full(x.shape, 0x7FFF, jnp.uint32) + lsb) & jnp.full(
        x.shape, 0xFFFF0000, jnp.uint32)
    return plsc.bitcast(r, jnp.float32)


def _sc_body(coord_hbm, samples_hbm, out_hbm, cvm, svm, ovm,
             cxb, cyb, czb, c2v, *, n, s, rows_per_w):
    wid = lax.axis_index("s") * _NC + lax.axis_index("c")
    row0 = wid * rows_per_w
    batch = row0 // s
    srow0 = row0 % s
    r2 = _RADIUS * _RADIUS

    pltpu.sync_copy(coord_hbm.at[batch], cvm)
    pltpu.sync_copy(samples_hbm.at[batch], svm)

    lanes = lax.iota(jnp.int32, 16)
    nv = jnp.full((_L,), n, jnp.int32)
    onesv = jnp.full((_L,), 1, jnp.int32)

    def pre_body(t, carry):
        jb = t * _L
        cx = cvm[0, pl.ds(jb, _L)]
        cy = cvm[1, pl.ds(jb, _L)]
        cz = cvm[2, pl.ds(jb, _L)]
        cxb[pl.ds(jb, _L)] = _bf(cx)
        cyb[pl.ds(jb, _L)] = _bf(cy)
        czb[pl.ds(jb, _L)] = _bf(cz)
        c2v[pl.ds(jb, _L)] = cx * cx + cy * cy + cz * cz
        return carry

    lax.fori_loop(0, n // _L, pre_body, 0)

    def load_row(i):
        sidx = jnp.full((_L,), srow0 + i, jnp.int32)
        sx = plsc.load_gather(svm, [sidx])
        sy = plsc.load_gather(svm, [sidx + jnp.full((_L,), s, jnp.int32)])
        sz = plsc.load_gather(svm, [sidx + jnp.full((_L,), 2 * s, jnp.int32)])
        s2 = sx * sx + sy * sy + sz * sz
        base = i * _NSAMPLE
        ovm[pl.ds(base, _L)] = nv
        ovm[pl.ds(base + _L, _L)] = nv
        basem1 = jnp.full((_L,), -1, jnp.int32) + base
        return (_bf(sx), _bf(sy), _bf(sz), s2, basem1)

    def fix_row(i):
        base = i * _NSAMPLE
        first = plsc.load_gather(ovm, [jnp.full((_L,), base, jnp.int32)])
        o1 = ovm[pl.ds(base, _L)]
        o2 = ovm[pl.ds(base + _L, _L)]
        ovm[pl.ds(base, _L)] = jnp.where(o1 == nv, first, o1)
        ovm[pl.ds(base + _L, _L)] = jnp.where(o2 == nv, first, o2)

    def pair_body(i2, carry):
        rowa = load_row(2 * i2)
        rowb = load_row(2 * i2 + 1)

        def step(jb, row, have_v, cxv, cyv, czv, c2, vals):
            sxb, syb, szb, s2, basem1 = row
            mm = sxb * cxv
            mm = mm + syb * cyv
            mm = mm + szb * czv
            d = (-2.0 * mm + s2) + c2
            msk = d <= r2
            mi = jnp.where(msk, 1, 0).astype(jnp.int32)
            ranks = plsc.cumsum(mi)
            pos = have_v + ranks
            wmask = msk & (pos <= _NSAMPLE)
            idx = pos + basem1
            plsc.store_scatter(ovm, [idx], vals, mask=wmask)
            cnt = plsc.all_reduce_population_count(msk)
            return have_v + cnt

        def cond(st):
            jb, _, _, done = st
            return jnp.logical_and(jb < n, jnp.logical_not(done))

        def body(st):
            jb, have_a, have_b, _ = st
            for u in range(2):
                jbu = jb + u * _L
                cxv = cxb[pl.ds(jbu, _L)]
                cyv = cyb[pl.ds(jbu, _L)]
                czv = czb[pl.ds(jbu, _L)]
                c2 = c2v[pl.ds(jbu, _L)]
                vals = jbu + lanes
                have_a = step(jbu, rowa, have_a, cxv, cyv, czv, c2, vals)
                have_b = step(jbu, rowb, have_b, cxv, cyv, czv, c2, vals)
            done = jnp.logical_and(jnp.any(have_a >= _NSAMPLE),
                                   jnp.any(have_b >= _NSAMPLE))
            return (jb + 2 * _L, have_a, have_b, done)

        lax.while_loop(cond, body,
                       (jnp.int32(0), jnp.zeros((_L,), jnp.int32),
                        jnp.zeros((_L,), jnp.int32), False))

        fix_row(2 * i2)
        fix_row(2 * i2 + 1)
        return carry

    lax.fori_loop(0, rows_per_w // 2, pair_body, 0)
    pltpu.sync_copy(ovm, out_hbm.at[pl.ds(row0 * _NSAMPLE,
                                          rows_per_w * _NSAMPLE)])


def kernel(coord, samples):
    b, n, _ = coord.shape
    s = samples.shape[1]
    rows_per_w = (b * s) // _NW

    coord_t = jnp.transpose(coord, (0, 2, 1))
    samples_t = jnp.transpose(samples, (0, 2, 1)).reshape(b, 3 * s)

    mesh = plsc.VectorSubcoreMesh(core_axis_name="c", subcore_axis_name="s",
                                  num_cores=_NC, num_subcores=_NS)
    out = pl.kernel(
        functools.partial(_sc_body, n=n, s=s, rows_per_w=rows_per_w),
        out_type=jax.ShapeDtypeStruct((b * s * _NSAMPLE,), jnp.int32),
        mesh=mesh,
        compiler_params=pltpu.CompilerParams(needs_layout_passes=False),
        scratch_types=[
            pltpu.VMEM((3, n), jnp.float32),
            pltpu.VMEM((3 * s,), jnp.float32),
            pltpu.VMEM((rows_per_w * _NSAMPLE,), jnp.int32),
            pltpu.VMEM((n,), jnp.float32),
            pltpu.VMEM((n,), jnp.float32),
            pltpu.VMEM((n,), jnp.float32),
            pltpu.VMEM((n,), jnp.float32),
        ],
    )(coord_t, samples_t)
    return out.reshape(b, s, _NSAMPLE)


# speedup vs baseline: 2.6656x; 1.4973x over previous
import functools

import jax
import jax.numpy as jnp
from jax import lax
from jax.experimental import pallas as pl
from jax.experimental.pallas import tpu as pltpu
from jax.experimental.pallas import tpu_sc as plsc

_RADIUS = 0.2
_NSAMPLE = 32
_NC = 2
_NS = 16
_NW = _NC * _NS
_L = 16
_G = 4


def _bf(x):
    u = plsc.bitcast(x, jnp.uint32)
    lsb = (u >> jnp.full(x.shape, 16, jnp.uint32)) & jnp.full(
        x.shape, 1, jnp.uint32)
    r = (u + jnp.full(x.shape, 0x7FFF, jnp.uint32) + lsb) & jnp.full(
        x.shape, 0xFFFF0000, jnp.uint32)
    return plsc.bitcast(r, jnp.float32)


def _sc_body(coord_hbm, samples_hbm, out_hbm, cvm, svm, ovm,
             cxb, cyb, czb, c2v, *, n, s, rows_per_w):
    wid = lax.axis_index("s") * _NC + lax.axis_index("c")
    row0 = wid * rows_per_w
    batch = row0 // s
    srow0 = row0 % s
    r2 = _RADIUS * _RADIUS

    pltpu.sync_copy(coord_hbm.at[batch], cvm)
    pltpu.sync_copy(samples_hbm.at[batch], svm)

    lanes = lax.iota(jnp.int32, 16)
    nv = jnp.full((_L,), n, jnp.int32)
    onesv = jnp.full((_L,), 1, jnp.int32)

    def pre_body(t, carry):
        jb = t * _L
        cx = cvm[0, pl.ds(jb, _L)]
        cy = cvm[1, pl.ds(jb, _L)]
        cz = cvm[2, pl.ds(jb, _L)]
        cxb[pl.ds(jb, _L)] = _bf(cx)
        cyb[pl.ds(jb, _L)] = _bf(cy)
        czb[pl.ds(jb, _L)] = _bf(cz)
        c2v[pl.ds(jb, _L)] = cx * cx + cy * cy + cz * cz
        return carry

    lax.fori_loop(0, n // _L, pre_body, 0)

    def load_row(i):
        sidx = jnp.full((_L,), srow0 + i, jnp.int32)
        sx = plsc.load_gather(svm, [sidx])
        sy = plsc.load_gather(svm, [sidx + jnp.full((_L,), s, jnp.int32)])
        sz = plsc.load_gather(svm, [sidx + jnp.full((_L,), 2 * s, jnp.int32)])
        s2 = sx * sx + sy * sy + sz * sz
        base = i * _NSAMPLE
        ovm[pl.ds(base, _L)] = nv
        ovm[pl.ds(base + _L, _L)] = nv
        basem1 = jnp.full((_L,), -1, jnp.int32) + base
        return (_bf(sx), _bf(sy), _bf(sz), s2, basem1)

    def fix_row(i):
        base = i * _NSAMPLE
        first = plsc.load_gather(ovm, [jnp.full((_L,), base, jnp.int32)])
        o1 = ovm[pl.ds(base, _L)]
        o2 = ovm[pl.ds(base + _L, _L)]
        ovm[pl.ds(base, _L)] = jnp.where(o1 == nv, first, o1)
        ovm[pl.ds(base + _L, _L)] = jnp.where(o2 == nv, first, o2)

    def pair_body(i2, carry):
        rows = [load_row(_G * i2 + g) for g in range(_G)]

        def step(jb, row, have_v, cxv, cyv, czv, c2, vals):
            sxb, syb, szb, s2, basem1 = row
            mm = sxb * cxv
            mm = mm + syb * cyv
            mm = mm + szb * czv
            d = (-2.0 * mm + s2) + c2
            msk = d <= r2
            mi = jnp.where(msk, 1, 0).astype(jnp.int32)
            ranks = plsc.cumsum(mi)
            pos = have_v + ranks
            wmask = msk & (pos <= _NSAMPLE)
            idx = pos + basem1
            plsc.store_scatter(ovm, [idx], vals, mask=wmask)
            cnt = plsc.all_reduce_population_count(msk)
            return have_v + cnt

        def cond(st):
            return jnp.logical_and(st[0] < n, jnp.logical_not(st[-1]))

        def body(st):
            jb = st[0]
            haves = list(st[1:1 + _G])
            for u in range(2):
                jbu = jb + u * _L
                cxv = cxb[pl.ds(jbu, _L)]
                cyv = cyb[pl.ds(jbu, _L)]
                czv = czb[pl.ds(jbu, _L)]
                c2 = c2v[pl.ds(jbu, _L)]
                vals = jbu + lanes
                for g in range(_G):
                    haves[g] = step(jbu, rows[g], haves[g],
                                    cxv, cyv, czv, c2, vals)
            done = jnp.any(haves[0] >= _NSAMPLE)
            for g in range(1, _G):
                done = jnp.logical_and(done, jnp.any(haves[g] >= _NSAMPLE))
            return (jb + 2 * _L, *haves, done)

        init = (jnp.int32(0),) + tuple(
            jnp.zeros((_L,), jnp.int32) for _ in range(_G)) + (False,)
        lax.while_loop(cond, body, init)

        for g in range(_G):
            fix_row(_G * i2 + g)
        return carry

    lax.fori_loop(0, rows_per_w // _G, pair_body, 0)
    pltpu.sync_copy(ovm, out_hbm.at[pl.ds(row0 * _NSAMPLE,
                                          rows_per_w * _NSAMPLE)])


def kernel(coord, samples):
    b, n, _ = coord.shape
    s = samples.shape[1]
    rows_per_w = (b * s) // _NW

    coord_t = jnp.transpose(coord, (0, 2, 1))
    samples_t = jnp.transpose(samples, (0, 2, 1)).reshape(b, 3 * s)

    mesh = plsc.VectorSubcoreMesh(core_axis_name="c", subcore_axis_name="s",
                                  num_cores=_NC, num_subcores=_NS)
    out = pl.kernel(
        functools.partial(_sc_body, n=n, s=s, rows_per_w=rows_per_w),
        out_type=jax.ShapeDtypeStruct((b * s * _NSAMPLE,), jnp.int32),
        mesh=mesh,
        compiler_params=pltpu.CompilerParams(needs_layout_passes=False),
        scratch_types=[
            pltpu.VMEM((3, n), jnp.float32),
            pltpu.VMEM((3 * s,), jnp.float32),
            pltpu.VMEM((rows_per_w * _NSAMPLE,), jnp.int32),
            pltpu.VMEM((n,), jnp.float32),
            pltpu.VMEM((n,), jnp.float32),
            pltpu.VMEM((n,), jnp.float32),
            pltpu.VMEM((n,), jnp.float32),
        ],
    )(coord_t, samples_t)
    return out.reshape(b, s, _NSAMPLE)


# speedup vs baseline: 3.0098x; 1.1291x over previous
import functools

import jax
import jax.numpy as jnp
from jax import lax
from jax.experimental import pallas as pl
from jax.experimental.pallas import tpu as pltpu
from jax.experimental.pallas import tpu_sc as plsc

_RADIUS = 0.2
_NSAMPLE = 32
_NC = 2
_NS = 16
_NW = _NC * _NS
_L = 16
_G = 8


def _bf(x):
    u = plsc.bitcast(x, jnp.uint32)
    lsb = (u >> jnp.full(x.shape, 16, jnp.uint32)) & jnp.full(
        x.shape, 1, jnp.uint32)
    r = (u + jnp.full(x.shape, 0x7FFF, jnp.uint32) + lsb) & jnp.full(
        x.shape, 0xFFFF0000, jnp.uint32)
    return plsc.bitcast(r, jnp.float32)


def _sc_body(coord_hbm, samples_hbm, out_hbm, cvm, svm, ovm,
             cxb, cyb, czb, c2v, *, n, s, rows_per_w):
    wid = lax.axis_index("s") * _NC + lax.axis_index("c")
    row0 = wid * rows_per_w
    batch = row0 // s
    srow0 = row0 % s
    r2 = _RADIUS * _RADIUS

    pltpu.sync_copy(coord_hbm.at[batch], cvm)
    pltpu.sync_copy(samples_hbm.at[batch], svm)

    lanes = lax.iota(jnp.int32, 16)
    nv = jnp.full((_L,), n, jnp.int32)
    onesv = jnp.full((_L,), 1, jnp.int32)

    def pre_body(t, carry):
        jb = t * _L
        cx = cvm[0, pl.ds(jb, _L)]
        cy = cvm[1, pl.ds(jb, _L)]
        cz = cvm[2, pl.ds(jb, _L)]
        cxb[pl.ds(jb, _L)] = _bf(cx)
        cyb[pl.ds(jb, _L)] = _bf(cy)
        czb[pl.ds(jb, _L)] = _bf(cz)
        c2v[pl.ds(jb, _L)] = cx * cx + cy * cy + cz * cz
        return carry

    lax.fori_loop(0, n // _L, pre_body, 0)

    def load_row(i):
        sidx = jnp.full((_L,), srow0 + i, jnp.int32)
        sx = plsc.load_gather(svm, [sidx])
        sy = plsc.load_gather(svm, [sidx + jnp.full((_L,), s, jnp.int32)])
        sz = plsc.load_gather(svm, [sidx + jnp.full((_L,), 2 * s, jnp.int32)])
        s2 = sx * sx + sy * sy + sz * sz
        base = i * _NSAMPLE
        ovm[pl.ds(base, _L)] = nv
        ovm[pl.ds(base + _L, _L)] = nv
        basem1 = jnp.full((_L,), -1, jnp.int32) + base
        return (_bf(sx), _bf(sy), _bf(sz), s2, basem1)

    def fix_row(i):
        base = i * _NSAMPLE
        first = plsc.load_gather(ovm, [jnp.full((_L,), base, jnp.int32)])
        o1 = ovm[pl.ds(base, _L)]
        o2 = ovm[pl.ds(base + _L, _L)]
        ovm[pl.ds(base, _L)] = jnp.where(o1 == nv, first, o1)
        ovm[pl.ds(base + _L, _L)] = jnp.where(o2 == nv, first, o2)

    def pair_body(i2, carry):
        rows = [load_row(_G * i2 + g) for g in range(_G)]

        def step(jb, row, have_v, cxv, cyv, czv, c2, vals):
            sxb, syb, szb, s2, basem1 = row
            mm = sxb * cxv
            mm = mm + syb * cyv
            mm = mm + szb * czv
            d = (-2.0 * mm + s2) + c2
            msk = d <= r2
            mi = jnp.where(msk, 1, 0).astype(jnp.int32)
            ranks = plsc.cumsum(mi)
            pos = have_v + ranks
            wmask = msk & (pos <= _NSAMPLE)
            idx = pos + basem1
            plsc.store_scatter(ovm, [idx], vals, mask=wmask)
            cnt = plsc.all_reduce_population_count(msk)
            return have_v + cnt

        def cond(st):
            return jnp.logical_and(st[0] < n, jnp.logical_not(st[-1]))

        def body(st):
            jb = st[0]
            haves = list(st[1:1 + _G])
            for u in range(2):
                jbu = jb + u * _L
                cxv = cxb[pl.ds(jbu, _L)]
                cyv = cyb[pl.ds(jbu, _L)]
                czv = czb[pl.ds(jbu, _L)]
                c2 = c2v[pl.ds(jbu, _L)]
                vals = jbu + lanes
                for g in range(_G):
                    haves[g] = step(jbu, rows[g], haves[g],
                                    cxv, cyv, czv, c2, vals)
            done = jnp.any(haves[0] >= _NSAMPLE)
            for g in range(1, _G):
                done = jnp.logical_and(done, jnp.any(haves[g] >= _NSAMPLE))
            return (jb + 2 * _L, *haves, done)

        init = (jnp.int32(0),) + tuple(
            jnp.zeros((_L,), jnp.int32) for _ in range(_G)) + (False,)
        lax.while_loop(cond, body, init)

        for g in range(_G):
            fix_row(_G * i2 + g)
        return carry

    lax.fori_loop(0, rows_per_w // _G, pair_body, 0)
    pltpu.sync_copy(ovm, out_hbm.at[pl.ds(row0 * _NSAMPLE,
                                          rows_per_w * _NSAMPLE)])


def kernel(coord, samples):
    b, n, _ = coord.shape
    s = samples.shape[1]
    rows_per_w = (b * s) // _NW

    coord_t = jnp.transpose(coord, (0, 2, 1))
    samples_t = jnp.transpose(samples, (0, 2, 1)).reshape(b, 3 * s)

    mesh = plsc.VectorSubcoreMesh(core_axis_name="c", subcore_axis_name="s",
                                  num_cores=_NC, num_subcores=_NS)
    out = pl.kernel(
        functools.partial(_sc_body, n=n, s=s, rows_per_w=rows_per_w),
        out_type=jax.ShapeDtypeStruct((b * s * _NSAMPLE,), jnp.int32),
        mesh=mesh,
        compiler_params=pltpu.CompilerParams(needs_layout_passes=False),
        scratch_types=[
            pltpu.VMEM((3, n), jnp.float32),
            pltpu.VMEM((3 * s,), jnp.float32),
            pltpu.VMEM((rows_per_w * _NSAMPLE,), jnp.int32),
            pltpu.VMEM((n,), jnp.float32),
            pltpu.VMEM((n,), jnp.float32),
            pltpu.VMEM((n,), jnp.float32),
            pltpu.VMEM((n,), jnp.float32),
        ],
    )(coord_t, samples_t)
    return out.reshape(b, s, _NSAMPLE)


# speedup vs baseline: 3.4784x; 1.1557x over previous
import functools

import jax
import jax.numpy as jnp
from jax import lax
from jax.experimental import pallas as pl
from jax.experimental.pallas import tpu as pltpu
from jax.experimental.pallas import tpu_sc as plsc

_RADIUS = 0.2
_NSAMPLE = 32
_NC = 2
_NS = 16
_NW = _NC * _NS
_L = 16
_G = 8


def _bf(x):
    u = plsc.bitcast(x, jnp.uint32)
    lsb = (u >> jnp.full(x.shape, 16, jnp.uint32)) & jnp.full(
        x.shape, 1, jnp.uint32)
    r = (u + jnp.full(x.shape, 0x7FFF, jnp.uint32) + lsb) & jnp.full(
        x.shape, 0xFFFF0000, jnp.uint32)
    return plsc.bitcast(r, jnp.float32)


def _sc_body(coord_hbm, samples_hbm, out_hbm, cvm, svm, ovm,
             cxb, cyb, czb, c2v, *, n, s, rows_per_w):
    wid = lax.axis_index("s") * _NC + lax.axis_index("c")
    row0 = wid * rows_per_w
    batch = row0 // s
    srow0 = row0 % s
    r2 = _RADIUS * _RADIUS

    pltpu.sync_copy(coord_hbm.at[batch], cvm)
    pltpu.sync_copy(samples_hbm.at[batch], svm)

    lanes = lax.iota(jnp.int32, 16)
    nv = jnp.full((_L,), n, jnp.int32)
    onesv = jnp.full((_L,), 1, jnp.int32)

    def pre_body(t, carry):
        jb = t * _L
        cx = cvm[0, pl.ds(jb, _L)]
        cy = cvm[1, pl.ds(jb, _L)]
        cz = cvm[2, pl.ds(jb, _L)]
        cxb[pl.ds(jb, _L)] = _bf(cx)
        cyb[pl.ds(jb, _L)] = _bf(cy)
        czb[pl.ds(jb, _L)] = _bf(cz)
        c2v[pl.ds(jb, _L)] = cx * cx + cy * cy + cz * cz
        return carry

    lax.fori_loop(0, n // _L, pre_body, 0)

    def load_row(i):
        sidx = jnp.full((_L,), srow0 + i, jnp.int32)
        sx = plsc.load_gather(svm, [sidx])
        sy = plsc.load_gather(svm, [sidx + jnp.full((_L,), s, jnp.int32)])
        sz = plsc.load_gather(svm, [sidx + jnp.full((_L,), 2 * s, jnp.int32)])
        s2 = sx * sx + sy * sy + sz * sz
        base = i * _NSAMPLE
        ovm[pl.ds(base, _L)] = nv
        ovm[pl.ds(base + _L, _L)] = nv
        basem1 = jnp.full((_L,), -1, jnp.int32) + base
        return (_bf(sx), _bf(sy), _bf(sz), s2, basem1)

    def fix_row(i):
        base = i * _NSAMPLE
        first = plsc.load_gather(ovm, [jnp.full((_L,), base, jnp.int32)])
        o1 = ovm[pl.ds(base, _L)]
        o2 = ovm[pl.ds(base + _L, _L)]
        ovm[pl.ds(base, _L)] = jnp.where(o1 == nv, first, o1)
        ovm[pl.ds(base + _L, _L)] = jnp.where(o2 == nv, first, o2)

    def pair_body(i2, carry):
        rows = [load_row(_G * i2 + g) for g in range(_G)]

        def step(jb, row, have_v, cxv, cyv, czv, c2, vals):
            sxb, syb, szb, s2, basem1 = row
            mm = sxb * cxv
            mm = mm + syb * cyv
            mm = mm + szb * czv
            d = (-2.0 * mm + s2) + c2
            msk = d <= r2
            mi = jnp.where(msk, 1, 0).astype(jnp.int32)
            ranks = plsc.cumsum(mi)
            pos = have_v + ranks
            wmask = msk & (pos <= _NSAMPLE)
            idx = pos + basem1
            plsc.store_scatter(ovm, [idx], vals, mask=wmask)
            cnt = plsc.all_reduce_population_count(msk)
            return have_v + cnt

        def cond(st):
            return jnp.logical_and(st[0] < n, jnp.logical_not(st[-1]))

        def body(st):
            jb = st[0]
            haves = list(st[1:1 + _G])
            for u in range(2):
                jbu = jb + u * _L
                cxv = cxb[pl.ds(jbu, _L)]
                cyv = cyb[pl.ds(jbu, _L)]
                czv = czb[pl.ds(jbu, _L)]
                c2 = c2v[pl.ds(jbu, _L)]
                vals = jbu + lanes
                for g in range(_G):
                    haves[g] = step(jbu, rows[g], haves[g],
                                    cxv, cyv, czv, c2, vals)
            done = jnp.any(haves[0] >= _NSAMPLE)
            for g in range(1, _G):
                done = jnp.logical_and(done, jnp.any(haves[g] >= _NSAMPLE))
            return (jb + 2 * _L, *haves, done)

        init = (jnp.int32(0),) + tuple(
            jnp.zeros((_L,), jnp.int32) for _ in range(_G)) + (False,)
        lax.while_loop(cond, body, init)

        for g in range(_G):
            fix_row(_G * i2 + g)
        return carry

    lax.fori_loop(0, rows_per_w // _G, pair_body, 0)
    pltpu.sync_copy(ovm, out_hbm.at[pl.ds(row0 * _NSAMPLE,
                                          rows_per_w * _NSAMPLE)])


def kernel(coord, samples):
    b, n, _ = coord.shape
    s = samples.shape[1]
    rows_per_w = (b * s) // _NW

    r = _RADIUS
    ov = jnp.clip(jnp.minimum(samples + r, 1.0) - jnp.maximum(samples - r, 0.0),
                  0.0, None)
    score = ov[..., 0] * ov[..., 1] * ov[..., 2]
    order = jnp.argsort(-score, axis=1)
    og = order.reshape(b, s // _G, _G)
    perm = jnp.concatenate(
        [og[:, 0::2].reshape(b, s // 2), og[:, 1::2].reshape(b, s // 2)],
        axis=1)
    inv = jnp.argsort(perm, axis=1)
    samples_p = jnp.take_along_axis(samples, perm[..., None], axis=1)

    coord_t = jnp.transpose(coord, (0, 2, 1))
    samples_t = jnp.transpose(samples_p, (0, 2, 1)).reshape(b, 3 * s)

    mesh = plsc.VectorSubcoreMesh(core_axis_name="c", subcore_axis_name="s",
                                  num_cores=_NC, num_subcores=_NS)
    out = pl.kernel(
        functools.partial(_sc_body, n=n, s=s, rows_per_w=rows_per_w),
        out_type=jax.ShapeDtypeStruct((b * s * _NSAMPLE,), jnp.int32),
        mesh=mesh,
        compiler_params=pltpu.CompilerParams(needs_layout_passes=False),
        scratch_types=[
            pltpu.VMEM((3, n), jnp.float32),
            pltpu.VMEM((3 * s,), jnp.float32),
            pltpu.VMEM((rows_per_w * _NSAMPLE,), jnp.int32),
            pltpu.VMEM((n,), jnp.float32),
            pltpu.VMEM((n,), jnp.float32),
            pltpu.VMEM((n,), jnp.float32),
            pltpu.VMEM((n,), jnp.float32),
        ],
    )(coord_t, samples_t)
    out = out.reshape(b, s, _NSAMPLE)
    return jnp.take_along_axis(out, inv[..., None], axis=1)


# speedup vs baseline: 3.6114x; 1.0382x over previous
import functools

import jax
import jax.numpy as jnp
from jax import lax
from jax.experimental import pallas as pl
from jax.experimental.pallas import tpu as pltpu
from jax.experimental.pallas import tpu_sc as plsc

_RADIUS = 0.2
_NSAMPLE = 32
_NC = 2
_NS = 16
_NW = _NC * _NS
_L = 16
_G = 8


def _bf(x):
    u = plsc.bitcast(x, jnp.uint32)
    lsb = (u >> jnp.full(x.shape, 16, jnp.uint32)) & jnp.full(
        x.shape, 1, jnp.uint32)
    r = (u + jnp.full(x.shape, 0x7FFF, jnp.uint32) + lsb) & jnp.full(
        x.shape, 0xFFFF0000, jnp.uint32)
    return plsc.bitcast(r, jnp.float32)


def _sc_body(coord_hbm, samples_hbm, out_hbm, cvm, svm, ovm,
             cxb, cyb, czb, c2v, *, n, s, rows_per_w):
    wid = lax.axis_index("s") * _NC + lax.axis_index("c")
    row0 = wid * rows_per_w
    batch = row0 // s
    srow0 = row0 % s
    r2 = _RADIUS * _RADIUS

    pltpu.sync_copy(coord_hbm.at[batch], cvm)
    pltpu.sync_copy(samples_hbm.at[batch], svm)

    lanes = lax.iota(jnp.int32, 16)
    nv = jnp.full((_L,), n, jnp.int32)
    onesv = jnp.full((_L,), 1, jnp.int32)

    def pre_body(t, carry):
        jb = t * _L
        cx = cvm[0, pl.ds(jb, _L)]
        cy = cvm[1, pl.ds(jb, _L)]
        cz = cvm[2, pl.ds(jb, _L)]
        cxb[pl.ds(jb, _L)] = _bf(cx)
        cyb[pl.ds(jb, _L)] = _bf(cy)
        czb[pl.ds(jb, _L)] = _bf(cz)
        c2v[pl.ds(jb, _L)] = cx * cx + cy * cy + cz * cz
        return carry

    lax.fori_loop(0, n // _L, pre_body, 0)

    def load_row(i):
        sidx = jnp.full((_L,), srow0 + i, jnp.int32)
        sx = plsc.load_gather(svm, [sidx])
        sy = plsc.load_gather(svm, [sidx + jnp.full((_L,), s, jnp.int32)])
        sz = plsc.load_gather(svm, [sidx + jnp.full((_L,), 2 * s, jnp.int32)])
        s2 = sx * sx + sy * sy + sz * sz
        base = i * _NSAMPLE
        ovm[pl.ds(base, _L)] = nv
        ovm[pl.ds(base + _L, _L)] = nv
        basem1 = jnp.full((_L,), -1, jnp.int32) + base
        return (_bf(sx), _bf(sy), _bf(sz), s2, basem1)

    def fix_row(i):
        base = i * _NSAMPLE
        first = plsc.load_gather(ovm, [jnp.full((_L,), base, jnp.int32)])
        o1 = ovm[pl.ds(base, _L)]
        o2 = ovm[pl.ds(base + _L, _L)]
        ovm[pl.ds(base, _L)] = jnp.where(o1 == nv, first, o1)
        ovm[pl.ds(base + _L, _L)] = jnp.where(o2 == nv, first, o2)

    def pair_body(i2, carry):
        rows = [load_row(_G * i2 + g) for g in range(_G)]

        def step(jb, row, have_v, cxv, cyv, czv, c2, vals):
            sxb, syb, szb, s2, basem1 = row
            mm = sxb * cxv
            mm = mm + syb * cyv
            mm = mm + szb * czv
            d = (-2.0 * mm + s2) + c2
            msk = d <= r2
            ranks = plsc.cumsum(onesv, mask=msk)
            pos = have_v + ranks
            wmask = msk & (pos <= _NSAMPLE)
            idx = pos + basem1
            plsc.store_scatter(ovm, [idx], vals, mask=wmask)
            cnt = plsc.all_reduce_population_count(msk)
            return have_v + cnt

        def cond(st):
            return jnp.logical_and(st[0] < n, jnp.logical_not(st[-1]))

        def body(st):
            jb = st[0]
            haves = list(st[1:1 + _G])
            for u in range(2):
                jbu = jb + u * _L
                cxv = cxb[pl.ds(jbu, _L)]
                cyv = cyb[pl.ds(jbu, _L)]
                czv = czb[pl.ds(jbu, _L)]
                c2 = c2v[pl.ds(jbu, _L)]
                vals = jbu + lanes
                for g in range(_G):
                    haves[g] = step(jbu, rows[g], haves[g],
                                    cxv, cyv, czv, c2, vals)
            done = jnp.any(haves[0] >= _NSAMPLE)
            for g in range(1, _G):
                done = jnp.logical_and(done, jnp.any(haves[g] >= _NSAMPLE))
            return (jb + 2 * _L, *haves, done)

        init = (jnp.int32(0),) + tuple(
            jnp.zeros((_L,), jnp.int32) for _ in range(_G)) + (False,)
        lax.while_loop(cond, body, init)

        for g in range(_G):
            fix_row(_G * i2 + g)
        return carry

    lax.fori_loop(0, rows_per_w // _G, pair_body, 0)
    pltpu.sync_copy(ovm, out_hbm.at[pl.ds(row0 * _NSAMPLE,
                                          rows_per_w * _NSAMPLE)])


def kernel(coord, samples):
    b, n, _ = coord.shape
    s = samples.shape[1]
    rows_per_w = (b * s) // _NW

    r = _RADIUS
    ov = jnp.clip(jnp.minimum(samples + r, 1.0) - jnp.maximum(samples - r, 0.0),
                  0.0, None)
    score = ov[..., 0] * ov[..., 1] * ov[..., 2]
    order = jnp.argsort(-score, axis=1)
    og = order.reshape(b, s // _G, _G)
    perm = jnp.concatenate(
        [og[:, 0::2].reshape(b, s // 2), og[:, 1::2].reshape(b, s // 2)],
        axis=1)
    inv = jnp.argsort(perm, axis=1)
    samples_p = jnp.take_along_axis(samples, perm[..., None], axis=1)

    coord_t = jnp.transpose(coord, (0, 2, 1))
    samples_t = jnp.transpose(samples_p, (0, 2, 1)).reshape(b, 3 * s)

    mesh = plsc.VectorSubcoreMesh(core_axis_name="c", subcore_axis_name="s",
                                  num_cores=_NC, num_subcores=_NS)
    out = pl.kernel(
        functools.partial(_sc_body, n=n, s=s, rows_per_w=rows_per_w),
        out_type=jax.ShapeDtypeStruct((b * s * _NSAMPLE,), jnp.int32),
        mesh=mesh,
        compiler_params=pltpu.CompilerParams(needs_layout_passes=False),
        scratch_types=[
            pltpu.VMEM((3, n), jnp.float32),
            pltpu.VMEM((3 * s,), jnp.float32),
            pltpu.VMEM((rows_per_w * _NSAMPLE,), jnp.int32),
            pltpu.VMEM((n,), jnp.float32),
            pltpu.VMEM((n,), jnp.float32),
            pltpu.VMEM((n,), jnp.float32),
            pltpu.VMEM((n,), jnp.float32),
        ],
    )(coord_t, samples_t)
    out = out.reshape(b, s, _NSAMPLE)
    return jnp.take_along_axis(out, inv[..., None], axis=1)
